# baseline (speedup 1.0000x reference)
import functools

import jax
import jax.numpy as jnp
from jax import lax
from jax.experimental import pallas as pl
from jax.experimental.pallas import tpu as pltpu
from jax.experimental.pallas import tpu_sc as plsc

B = 16384
D = 64
NC = 2
NS = 16
NW = NC * NS
BPW = B // NW
CHUNK = 128
NCHUNK = BPW // CHUNK

_mesh = plsc.VectorSubcoreMesh(core_axis_name="c", subcore_axis_name="s")


@functools.partial(
    pl.kernel,
    mesh=_mesh,
    out_type=jax.ShapeDtypeStruct((B,), jnp.float32),
    compiler_params=pltpu.CompilerParams(
        needs_layout_passes=False, use_tc_tiling_on_sc=False),
    scratch_types=[
        pltpu.VMEM((NCHUNK, CHUNK), jnp.int32),
        pltpu.VMEM((NCHUNK, CHUNK), jnp.int32),
        pltpu.VMEM((NCHUNK, CHUNK), jnp.int32),
        pltpu.VMEM((BPW, D), jnp.float32),
        pltpu.VMEM((BPW, D), jnp.float32),
        pltpu.VMEM((BPW, D), jnp.float32),
        pltpu.VMEM((BPW,), jnp.float32),
        pltpu.SemaphoreType.DMA((NCHUNK,)),
    ],
)
def _bt_norm_kernel(i_hbm, j_hbm, k_hbm, u_hbm, v_hbm, out_hbm,
                    iv, jv, kv, uv, vjv, vkv, outv, sems):
    wid = lax.axis_index("s") * NC + lax.axis_index("c")
    base = wid * BPW

    for c in range(NCHUNK):
        off = pl.ds(base + c * CHUNK, CHUNK)
        pltpu.sync_copy(i_hbm.at[off], iv.at[c])
        pltpu.sync_copy(j_hbm.at[off], jv.at[c])
        pltpu.sync_copy(k_hbm.at[off], kv.at[c])

    copies = []
    for c in range(NCHUNK):
        dst = pl.ds(c * CHUNK, CHUNK)
        copies.append((
            pltpu.async_copy(u_hbm.at[iv.at[c]], uv.at[dst], sems.at[c]),
            pltpu.async_copy(v_hbm.at[jv.at[c]], vjv.at[dst], sems.at[c]),
            pltpu.async_copy(v_hbm.at[kv.at[c]], vkv.at[dst], sems.at[c]),
        ))

    lane = lax.iota(jnp.int32, 16)

    def group(g, carry):
        rows = g * 16 + lane
        accj = jnp.zeros((16,), jnp.float32)
        acck = jnp.zeros((16,), jnp.float32)
        for d in range(D):
            col = (lane + d) & (D - 1)
            uval = plsc.load_gather(uv, [rows, col])
            jval = plsc.load_gather(vjv, [rows, col])
            kval = plsc.load_gather(vkv, [rows, col])
            dj = uval - jval
            dk = uval - kval
            accj = accj + dj * dj
            acck = acck + dk * dk
        x = accj - acck
        outv[pl.ds(g * 16, 16)] = 1.0 / (1.0 + jnp.exp(x))
        return carry

    for c in range(NCHUNK):
        for cp in copies[c]:
            cp.wait()
        lax.fori_loop(c * (CHUNK // 16), (c + 1) * (CHUNK // 16), group, 0)

    pltpu.sync_copy(outv, out_hbm.at[pl.ds(base, BPW)])


def kernel(i, j, k, u_weight, v_weight):
    return _bt_norm_kernel(
        i.astype(jnp.int32), j.astype(jnp.int32), k.astype(jnp.int32),
        u_weight, v_weight)


# speedup vs baseline: 1.0711x; 1.0711x over previous
import functools

import jax
import jax.numpy as jnp
from jax import lax
from jax.experimental import pallas as pl
from jax.experimental.pallas import tpu as pltpu
from jax.experimental.pallas import tpu_sc as plsc

B = 16384
D = 64
NC = 2
NS = 16
NW = NC * NS
BPW = B // NW
CHUNK = 128
NCHUNK = BPW // CHUNK
GPC = CHUNK // 16

_mesh = plsc.VectorSubcoreMesh(core_axis_name="c", subcore_axis_name="s")


@functools.partial(
    pl.kernel,
    mesh=_mesh,
    out_type=jax.ShapeDtypeStruct((B,), jnp.float32),
    compiler_params=pltpu.CompilerParams(needs_layout_passes=False),
    scratch_types=[
        pltpu.VMEM((NCHUNK, CHUNK), jnp.int32),
        pltpu.VMEM((NCHUNK, CHUNK), jnp.int32),
        pltpu.VMEM((NCHUNK, CHUNK), jnp.int32),
        pltpu.VMEM((NCHUNK, CHUNK), jnp.int32),
        pltpu.VMEM((NCHUNK, CHUNK), jnp.int32),
        pltpu.VMEM((NCHUNK, CHUNK), jnp.int32),
        pltpu.VMEM((2 * CHUNK, 2 * D), jnp.float32),
        pltpu.VMEM((2 * CHUNK, 2 * D), jnp.float32),
        pltpu.VMEM((2 * CHUNK, 2 * D), jnp.float32),
        pltpu.VMEM((BPW,), jnp.float32),
        pltpu.SemaphoreType.DMA((NCHUNK,)),
    ],
)
def _bt_norm_kernel(i_hbm, j_hbm, k_hbm, u_hbm, v_hbm, out_hbm,
                    iv, jv, kv, ivp, jvp, kvp, uv, vjv, vkv, outv, sems):
    wid = lax.axis_index("s") * NC + lax.axis_index("c")
    base = wid * BPW

    for c in range(NCHUNK):
        off = pl.ds(base + c * CHUNK, CHUNK)
        pltpu.sync_copy(i_hbm.at[off], iv.at[c])
        pltpu.sync_copy(j_hbm.at[off], jv.at[c])
        pltpu.sync_copy(k_hbm.at[off], kv.at[c])

    for c in range(NCHUNK):
        for t in range(GPC):
            s = pl.ds(t * 16, 16)
            ivp[c, s] = lax.shift_right_logical(iv[c, s], 1)
            jvp[c, s] = lax.shift_right_logical(jv[c, s], 1)
            kvp[c, s] = lax.shift_right_logical(kv[c, s], 1)

    def issue(c):
        half = pl.ds((c % 2) * CHUNK, CHUNK)
        return (
            pltpu.async_copy(u_hbm.at[ivp.at[c]], uv.at[half], sems.at[c]),
            pltpu.async_copy(v_hbm.at[jvp.at[c]], vjv.at[half], sems.at[c]),
            pltpu.async_copy(v_hbm.at[kvp.at[c]], vkv.at[half], sems.at[c]),
        )

    copies = {c: issue(c) for c in range(2)}

    lane = lax.iota(jnp.int32, 16)

    def make_group(c):
        def group(t, carry):
            rows = (c % 2) * CHUNK + t * 16 + lane
            s = pl.ds(t * 16, 16)
            pu = (iv[c, s] & 1) * D
            pj = (jv[c, s] & 1) * D
            pk = (kv[c, s] & 1) * D
            accj = jnp.zeros((16,), jnp.float32)
            acck = jnp.zeros((16,), jnp.float32)
            for d in range(D):
                sk = (lane + d) & (D - 1)
                uval = plsc.load_gather(uv, [rows, pu + sk])
                jval = plsc.load_gather(vjv, [rows, pj + sk])
                kval = plsc.load_gather(vkv, [rows, pk + sk])
                dj = uval - jval
                dk = uval - kval
                accj = accj + dj * dj
                acck = acck + dk * dk
            x = accj - acck
            outv[pl.ds(c * CHUNK + t * 16, 16)] = 1.0 / (1.0 + jnp.exp(x))
            return carry
        return group

    for c in range(NCHUNK):
        for cp in copies.pop(c):
            cp.wait()
        lax.fori_loop(0, GPC, make_group(c), 0)
        if c + 2 < NCHUNK:
            copies[c + 2] = issue(c + 2)

    pltpu.sync_copy(outv, out_hbm.at[pl.ds(base, BPW)])


def kernel(i, j, k, u_weight, v_weight):
    u2 = u_weight.reshape(u_weight.shape[0] // 2, 2 * D)
    v2 = v_weight.reshape(v_weight.shape[0] // 2, 2 * D)
    return _bt_norm_kernel(
        i.astype(jnp.int32), j.astype(jnp.int32), k.astype(jnp.int32),
        u2, v2)


# speedup vs baseline: 1.4097x; 1.3161x over previous
import functools

import jax
import jax.numpy as jnp
from jax import lax
from jax.experimental import pallas as pl
from jax.experimental.pallas import tpu as pltpu
from jax.experimental.pallas import tpu_sc as plsc

B = 16384
D = 64
NC = 2
NS = 16
NW = NC * NS
BPW = B // NW
CHUNK = 128
NCHUNK = BPW // CHUNK
GPC = CHUNK // 16

_mesh = plsc.VectorSubcoreMesh(core_axis_name="c", subcore_axis_name="s")


@functools.partial(
    pl.kernel,
    mesh=_mesh,
    out_type=jax.ShapeDtypeStruct((B,), jnp.float32),
    compiler_params=pltpu.CompilerParams(needs_layout_passes=False),
    scratch_types=[
        pltpu.VMEM((BPW,), jnp.int32),
        pltpu.VMEM((BPW,), jnp.int32),
        pltpu.VMEM((BPW,), jnp.int32),
        pltpu.VMEM((BPW // 2, 128), jnp.float32),
        pltpu.VMEM((BPW // 2, 128), jnp.float32),
        pltpu.VMEM((BPW // 2, 128), jnp.float32),
        pltpu.VMEM((BPW,), jnp.float32),
        pltpu.SemaphoreType.DMA((NCHUNK,)),
    ],
)
def _bt_norm_kernel(i_hbm, j_hbm, k_hbm, u_hbm, v_hbm, out_hbm,
                    iv, jv, kv, uv, vjv, vkv, outv, sems):
    wid = lax.axis_index("s") * NC + lax.axis_index("c")
    base = wid * BPW
    pltpu.sync_copy(i_hbm.at[pl.ds(base, BPW)], iv)
    pltpu.sync_copy(j_hbm.at[pl.ds(base, BPW)], jv)
    pltpu.sync_copy(k_hbm.at[pl.ds(base, BPW)], kv)

    def issue16(t, carry):
        ivec = iv[pl.ds(t * 16, 16)]
        jvec = jv[pl.ds(t * 16, 16)]
        kvec = kv[pl.ds(t * 16, 16)]
        c = t // (CHUNK // 16)
        for l in range(16):
            dst = (t * 8 + (l // 2), pl.ds((l % 2) * D, D))
            pltpu.async_copy(u_hbm.at[ivec[l]], uv.at[dst], sems.at[c])
            pltpu.async_copy(v_hbm.at[jvec[l]], vjv.at[dst], sems.at[c])
            pltpu.async_copy(v_hbm.at[kvec[l]], vkv.at[dst], sems.at[c])
        return carry

    lax.fori_loop(0, BPW // 16, issue16, 0)

    lane = lax.iota(jnp.int32, 16)
    halfrow = lax.shift_right_logical(lane, 1)
    colbase = (lane & 1) * D

    def group(g, carry):
        rows2 = g * 8 + halfrow
        accj = jnp.zeros((16,), jnp.float32)
        acck = jnp.zeros((16,), jnp.float32)
        for d in range(D):
            col = colbase + ((lane + d) & (D - 1))
            uval = plsc.load_gather(uv, [rows2, col])
            jval = plsc.load_gather(vjv, [rows2, col])
            kval = plsc.load_gather(vkv, [rows2, col])
            dj = uval - jval
            dk = uval - kval
            accj = accj + dj * dj
            acck = acck + dk * dk
        x = accj - acck
        outv[pl.ds(g * 16, 16)] = 1.0 / (1.0 + jnp.exp(x))
        return carry

    for c in range(NCHUNK):
        def drain(t, carry):
            pltpu.make_async_copy(i_hbm.at[pl.ds(0, BPW)], iv, sems.at[c]).wait()
            return carry
        lax.fori_loop(0, (3 * CHUNK * D) // BPW, drain, 0)
        lax.fori_loop(c * GPC, (c + 1) * GPC, group, 0)

    pltpu.sync_copy(outv, out_hbm.at[pl.ds(base, BPW)])


def kernel(i, j, k, u_weight, v_weight):
    return _bt_norm_kernel(
        i.astype(jnp.int32), j.astype(jnp.int32), k.astype(jnp.int32),
        u_weight, v_weight)
